# baseline (speedup 1.0000x reference)
import functools
import math

import jax
import jax.numpy as jnp
from jax import lax
from jax.experimental import pallas as pl
from jax.experimental.pallas import tpu as pltpu
from jax.experimental.pallas import tpu_sc as plsc

D_MODEL = 64
SCALE = math.sqrt(D_MODEL)
NUM_CORES = 2
NUM_SUBCORES = 16
NUM_WORKERS = NUM_CORES * NUM_SUBCORES
CHUNK = 128
LANES = 16


@functools.partial(jax.jit, static_argnames=("n_chunks_w",))
def _emb_call(idx, table, n_chunks_w):
    b_per_w = n_chunks_w * CHUNK
    batch = NUM_WORKERS * b_per_w

    mesh = plsc.VectorSubcoreMesh(core_axis_name="c", subcore_axis_name="s")

    @functools.partial(
        pl.kernel,
        out_type=jax.ShapeDtypeStruct((batch, D_MODEL), jnp.float32),
        mesh=mesh,
        compiler_params=pltpu.CompilerParams(use_tc_tiling_on_sc=False),
        scratch_types=[
            pltpu.VMEM((n_chunks_w, CHUNK), jnp.int32),
            pltpu.VMEM((CHUNK, D_MODEL), jnp.float32),
            pltpu.VMEM((CHUNK, D_MODEL), jnp.float32),
            pltpu.SemaphoreType.DMA,
            pltpu.SemaphoreType.DMA,
            pltpu.SemaphoreType.DMA,
            pltpu.SemaphoreType.DMA,
        ],
    )
    def kern(idx_hbm, table_hbm, out_hbm, idx_v, rows0, rows1,
             gsem0, gsem1, osem0, osem1):
        wid = lax.axis_index("s") * NUM_CORES + lax.axis_index("c")
        base = wid * b_per_w

        pltpu.sync_copy(idx_hbm.at[wid], idx_v)

        rows = (rows0, rows1)
        gsem = (gsem0, gsem1)
        osem = (osem0, osem1)

        def start_gather(j, b):
            pltpu.async_copy(table_hbm.at[idx_v.at[j]], rows[b], gsem[b])

        def wait_gather(j, b):
            pltpu.make_async_copy(
                table_hbm.at[idx_v.at[j]], rows[b], gsem[b]).wait()

        def out_slice(j):
            return out_hbm.at[pl.ds(base + j * CHUNK, CHUNK)]

        def start_out(j, b):
            pltpu.async_copy(rows[b], out_slice(j), osem[b])

        def wait_out(j, b):
            pltpu.make_async_copy(rows[b], out_slice(j), osem[b]).wait()

        def scale(b):
            rv = rows[b]

            @pl.loop(0, CHUNK, unroll=4)
            def _(r):
                for c4 in range(D_MODEL // LANES):
                    sl = pl.ds(c4 * LANES, LANES)
                    rv[r, sl] = rv[r, sl] * SCALE

        start_gather(0, 0)

        @pl.loop(0, n_chunks_w, step=2)
        def _(j0):
            for t in range(2):
                j = j0 + t
                other = 1 - t
                if t == 0:
                    @pl.when(j0 > 0)
                    def _():
                        wait_out(j - 1, other)
                else:
                    wait_out(j - 1, other)

                @pl.when(j + 1 < n_chunks_w)
                def _():
                    start_gather(j + 1, other)

                wait_gather(j, t)
                scale(t)
                start_out(j, t)

        wait_out(n_chunks_w - 1, 1)

    return kern(idx, table)


def kernel(x, table):
    batch = x.shape[0] * x.shape[1]
    n_chunks_w = batch // (NUM_WORKERS * CHUNK)
    idx = x.astype(jnp.int32).reshape(NUM_WORKERS, n_chunks_w, CHUNK)
    out = _emb_call(idx, table, n_chunks_w)
    return out.reshape(x.shape[0], x.shape[1], D_MODEL)
